# baseline (speedup 1.0000x reference)
import jax
import jax.numpy as jnp
from jax.experimental import pallas as pl
from jax.experimental.pallas import tpu as pltpu

N = 8192
D = 64
BM = 256
NQ = N // 4


def _pool_kernel(a_ref, x_ref, xr_ref, o_ref, xb_scr):
    i = pl.program_id(0)
    q = pl.program_id(1)

    @pl.when((i == 0) & (q == 0))
    def _cast_x():
        xb_scr[...] = x_ref[...].astype(jnp.bfloat16)

    partial = jnp.dot(a_ref[...].astype(jnp.bfloat16),
                      xb_scr[pl.ds(q * NQ, NQ), :],
                      preferred_element_type=jnp.float32)

    @pl.when(q == 0)
    def _init():
        o_ref[...] = xr_ref[...] + partial

    @pl.when(q > 0)
    def _accum():
        o_ref[...] += partial


def kernel(x, adj):
    return pl.pallas_call(
        _pool_kernel,
        grid=(N // BM, N // NQ),
        in_specs=[
            pl.BlockSpec((BM, NQ), lambda i, q: (i, q)),
            pl.BlockSpec((N, D), lambda i, q: (0, 0)),
            pl.BlockSpec((BM, D), lambda i, q: (i, 0)),
        ],
        out_specs=pl.BlockSpec((BM, D), lambda i, q: (i, 0)),
        out_shape=jax.ShapeDtypeStruct((N, D), jnp.float32),
        scratch_shapes=[pltpu.VMEM((N, D), jnp.bfloat16)],
        compiler_params=pltpu.CompilerParams(
            dimension_semantics=("arbitrary", "arbitrary"),
        ),
    )(adj, x, x)


# speedup vs baseline: 1.3196x; 1.3196x over previous
import jax
import jax.numpy as jnp
from jax.experimental import pallas as pl
from jax.experimental.pallas import tpu as pltpu

N = 8192
D = 64
BM = 128
NQ = N // 4


def _pool_kernel(a0, a1, a2, a3, x_ref, xr_ref, o_ref, xb_scr):
    i = pl.program_id(0)

    @pl.when(i == 0)
    def _cast_x():
        xb_scr[...] = x_ref[...].astype(jnp.bfloat16)

    acc = jnp.dot(a0[...].astype(jnp.bfloat16), xb_scr[0 * NQ:1 * NQ, :],
                  preferred_element_type=jnp.float32)
    acc += jnp.dot(a1[...].astype(jnp.bfloat16), xb_scr[1 * NQ:2 * NQ, :],
                   preferred_element_type=jnp.float32)
    acc += jnp.dot(a2[...].astype(jnp.bfloat16), xb_scr[2 * NQ:3 * NQ, :],
                   preferred_element_type=jnp.float32)
    acc += jnp.dot(a3[...].astype(jnp.bfloat16), xb_scr[3 * NQ:4 * NQ, :],
                   preferred_element_type=jnp.float32)
    o_ref[...] = xr_ref[...] + acc


def kernel(x, adj):
    return pl.pallas_call(
        _pool_kernel,
        grid=(N // BM,),
        in_specs=[
            pl.BlockSpec((BM, NQ), lambda i: (i, 0)),
            pl.BlockSpec((BM, NQ), lambda i: (i, 1)),
            pl.BlockSpec((BM, NQ), lambda i: (i, 2)),
            pl.BlockSpec((BM, NQ), lambda i: (i, 3)),
            pl.BlockSpec((N, D), lambda i: (0, 0)),
            pl.BlockSpec((BM, D), lambda i: (i, 0)),
        ],
        out_specs=pl.BlockSpec((BM, D), lambda i: (i, 0)),
        out_shape=jax.ShapeDtypeStruct((N, D), jnp.float32),
        scratch_shapes=[pltpu.VMEM((N, D), jnp.bfloat16)],
        compiler_params=pltpu.CompilerParams(
            dimension_semantics=("arbitrary",),
        ),
    )(adj, adj, adj, adj, x, x)
